# baseline (speedup 1.0000x reference)
import functools

import jax
import jax.numpy as jnp
from jax import lax
from jax.experimental import pallas as pl
from jax.experimental.pallas import tpu as pltpu
from jax.experimental.pallas import tpu_sc as plsc

_SUB = 128
_NB = 5
_CB = 128


def _worker_id():
    return lax.axis_index("s") * plsc.get_sparse_core_info().num_cores + lax.axis_index(
        "c"
    )


@jax.jit
def _sc_pack(table_t3):
    tr, fr, vocab = table_t3.shape
    emb = tr * fr
    info = plsc.get_sparse_core_info()
    nc, ns = info.num_cores, info.num_subcores
    nw = nc * ns
    qrows_blk = _CB // 2
    n_glob = vocab // _CB
    tail_cols = vocab - n_glob * _CB
    n_full, extra = divmod(n_glob, nw)
    trips = (n_full + (1 if extra else 0) + 1) // 2

    mesh = plsc.VectorSubcoreMesh(core_axis_name="c", subcore_axis_name="s")

    @functools.partial(
        pl.kernel,
        out_type=jax.ShapeDtypeStruct((vocab // 2, 2 * emb), jnp.float32),
        mesh=mesh,
        scratch_types=[
            [pltpu.VMEM((tr, fr, _CB), jnp.float32)] * 2,
            [pltpu.VMEM((qrows_blk, 2 * emb), jnp.float32)] * 2,
            [pltpu.SemaphoreType.DMA] * 2,
            [pltpu.SemaphoreType.DMA] * 2,
        ],
        compiler_params=pltpu.CompilerParams(
            use_tc_tiling_on_sc=True, needs_layout_passes=False
        ),
    )
    def body(tbl_hbm, tail_hbm, out_hbm, slabs, outs, isems, osems):
        wid = _worker_id()
        n_w = n_full + (wid < extra)
        lanes = lax.iota(jnp.int32, 16)
        tr16 = lanes >> 3
        fr16 = lanes & 7

        def col0(l):
            return (wid + l * nw) * _CB

        def fire(l, b):
            pltpu.async_copy(
                tbl_hbm.at[:, :, pl.ds(col0(l), _CB)], slabs[b], isems[b]
            )

        def compute_and_write(l, b):
            pltpu.make_async_copy(
                tbl_hbm.at[:, :, pl.ds(col0(l), _CB)], slabs[b], isems[b]
            ).wait()

            def row(q, carry):
                for g in range(emb // 16):
                    t16 = tr16 + 2 * g
                    vals0 = plsc.load_gather(
                        slabs[b], [t16, fr16, jnp.full((16,), 0, jnp.int32) + 2 * q]
                    )
                    outs[b][q, pl.ds(g * 16, 16)] = vals0
                    vals1 = plsc.load_gather(
                        slabs[b],
                        [t16, fr16, jnp.full((16,), 1, jnp.int32) + 2 * q],
                    )
                    outs[b][q, pl.ds(emb + g * 16, 16)] = vals1
                return carry

            lax.fori_loop(0, qrows_blk, row, 0)
            pltpu.async_copy(
                outs[b],
                out_hbm.at[pl.ds((wid + l * nw) * qrows_blk, qrows_blk)],
                osems[b],
            )

        def drain_write(b):
            pltpu.make_async_copy(
                outs[b], out_hbm.at[pl.ds(0, qrows_blk)], osems[b]
            ).wait()

        fire(0, 0)
        fire(1, 1)

        def step(t, carry):
            for b in range(2):
                l = t * 2 + b

                @pl.when(l < n_w)
                def _work():
                    compute_and_write(l, b)

                @pl.when(l + 2 < n_w)
                def _refill():
                    drain_write(b)
                    fire(l + 2, b)

            return carry

        lax.fori_loop(0, trips, step, 0)
        drain_write(0)
        drain_write(1)

        if tail_cols:
            tq = tail_cols // 2

            @pl.when(wid == nw - 1)
            def _tail():
                pltpu.sync_copy(tail_hbm, slabs[0])

                def row(q, carry):
                    for g in range(emb // 16):
                        t16 = tr16 + 2 * g
                        vals0 = plsc.load_gather(
                            slabs[0],
                            [t16, fr16, jnp.full((16,), 0, jnp.int32) + 2 * q],
                        )
                        outs[0][q, pl.ds(g * 16, 16)] = vals0
                        vals1 = plsc.load_gather(
                            slabs[0],
                            [t16, fr16, jnp.full((16,), 1, jnp.int32) + 2 * q],
                        )
                        outs[0][q, pl.ds(emb + g * 16, 16)] = vals1
                    return carry

                lax.fori_loop(0, tq, row, 0)
                pltpu.sync_copy(
                    outs[0].at[pl.ds(0, tq)],
                    out_hbm.at[pl.ds(n_glob * qrows_blk, tq)],
                )

    tail = table_t3[:, :, n_glob * _CB :]
    tail_padded = jnp.pad(tail, ((0, 0), (0, 0), (0, _CB - tail_cols)))
    return body(table_t3, tail_padded)


@jax.jit
def _sc_embed(idx_t, table_pairs):
    hist, batch = idx_t.shape
    emb = table_pairs.shape[1] // 2
    tr, fr = emb // 8, 8
    info = plsc.get_sparse_core_info()
    nc, ns = info.num_cores, info.num_subcores
    nw = nc * ns
    n_bblk = batch // _SUB
    assert n_bblk == nw and hist % _NB == 0

    mesh = plsc.VectorSubcoreMesh(core_axis_name="c", subcore_axis_name="s")

    @functools.partial(
        pl.kernel,
        out_type=jax.ShapeDtypeStruct((hist, tr, n_bblk, fr, _SUB), jnp.float32),
        mesh=mesh,
        scratch_types=[
            pltpu.VMEM((hist, _SUB), jnp.int32),
            [pltpu.VMEM((_SUB,), jnp.int32)] * _NB,
            pltpu.VMEM((_SUB,), jnp.int32),
            [pltpu.VMEM((_SUB, 2 * emb), jnp.float32)] * _NB,
            [pltpu.VMEM((tr, fr, _SUB), jnp.float32)] * 2,
            [pltpu.SemaphoreType.DMA] * _NB,
            [pltpu.SemaphoreType.DMA] * 2,
        ],
        compiler_params=pltpu.CompilerParams(
            use_tc_tiling_on_sc=True, needs_layout_passes=False
        ),
    )
    def body(idx_hbm, tbl_hbm, out_hbm, idx_v, q_vs, hv, stages, outbs, gsems, wsems):
        wid = _worker_id()
        pltpu.sync_copy(idx_hbm.at[:, pl.ds(wid * _SUB, _SUB)], idx_v)
        lanes = lax.iota(jnp.int32, 16)

        def fire(h, b):
            for g in range(_SUB // 16):
                sl = pl.ds(g * 16, 16)
                q_vs[b][sl] = lax.shift_right_logical(idx_v[h, sl], 1)
            pltpu.async_copy(tbl_hbm.at[q_vs[b]], stages[b], gsems[b])

        def wait_outb(h, ob):
            pltpu.make_async_copy(
                outbs[ob], out_hbm.at[h, :, wid], wsems[ob]
            ).wait()

        def drain(h, b, ob):
            pltpu.make_async_copy(
                tbl_hbm.at[q_vs[b]], stages[b], gsems[b]
            ).wait()
            for g in range(_SUB // 16):
                sl = pl.ds(g * 16, 16)
                hv[sl] = lax.shift_left(jnp.bitwise_and(idx_v[h, sl], 1), 6)

            def feat(f, carry):
                for g in range(_SUB // 16):
                    p16 = lanes + g * 16
                    cols = hv[pl.ds(g * 16, 16)] + f
                    vals = plsc.load_gather(stages[b], [p16, cols])
                    outbs[ob][f >> 3, f & 7, pl.ds(g * 16, 16)] = vals
                return carry

            lax.fori_loop(0, emb, feat, 0)
            pltpu.async_copy(outbs[ob], out_hbm.at[h, :, wid], wsems[ob])

        for b in range(_NB):
            fire(b, b)

        def step(t, carry):
            for u in range(2):
                for b in range(_NB):
                    k = u * _NB + b
                    h = t * (2 * _NB) + k
                    ob = k % 2

                    @pl.when(h >= 2)
                    def _wait_prev_outb():
                        wait_outb(h - 2, ob)

                    drain(h, b, ob)
                    nxt = h + _NB

                    @pl.when(nxt < hist)
                    def _refill():
                        fire(nxt, b)

            return carry

        lax.fori_loop(0, hist // (2 * _NB), step, 0)
        wait_outb(hist - 2, (2 * _NB - 2) % 2)
        wait_outb(hist - 1, (2 * _NB - 1) % 2)

    return body(idx_t, table_pairs)


def kernel(input_ids, embedding_weight):
    batch, hist = input_ids.shape
    vocab, emb = embedding_weight.shape

    table_t3 = embedding_weight.T.reshape(emb // 8, 8, vocab)
    table_pairs = _sc_pack(table_t3)
    idx_t = input_ids.astype(jnp.int32).T
    out5 = _sc_embed(idx_t, table_pairs)
    return out5.transpose(2, 4, 0, 1, 3).reshape(batch, hist, emb)


# speedup vs baseline: 1.8935x; 1.8935x over previous
import functools

import jax
import jax.numpy as jnp
from jax import lax
from jax.experimental import pallas as pl
from jax.experimental.pallas import tpu as pltpu
from jax.experimental.pallas import tpu_sc as plsc

_SUB = 128
_NB = 5
_CB = 128


def _worker_id():
    return lax.axis_index("s") * plsc.get_sparse_core_info().num_cores + lax.axis_index(
        "c"
    )


@jax.jit
def _sc_pack(table_t3):
    tr, fr, vocab = table_t3.shape
    emb = tr * fr
    info = plsc.get_sparse_core_info()
    nc, ns = info.num_cores, info.num_subcores
    nw = nc * ns
    qrows_blk = _CB // 2
    n_glob = vocab // _CB
    tail_cols = vocab - n_glob * _CB
    n_full, extra = divmod(n_glob, nw)
    trips = (n_full + (1 if extra else 0) + 1) // 2

    mesh = plsc.VectorSubcoreMesh(core_axis_name="c", subcore_axis_name="s")

    @functools.partial(
        pl.kernel,
        out_type=jax.ShapeDtypeStruct((vocab // 2, 2 * emb), jnp.float32),
        mesh=mesh,
        scratch_types=[
            [pltpu.VMEM((tr, fr, _CB), jnp.float32)] * 2,
            [pltpu.VMEM((qrows_blk, 2 * emb), jnp.float32)] * 2,
            [pltpu.SemaphoreType.DMA] * 2,
            [pltpu.SemaphoreType.DMA] * 2,
        ],
        compiler_params=pltpu.CompilerParams(
            use_tc_tiling_on_sc=True, needs_layout_passes=False
        ),
    )
    def body(tbl_hbm, tail_hbm, out_hbm, slabs, outs, isems, osems):
        wid = _worker_id()
        n_w = n_full + (wid < extra)
        lanes = lax.iota(jnp.int32, 16)
        tr16 = lanes >> 3
        fr16 = lanes & 7

        def col0(l):
            return (wid + l * nw) * _CB

        def fire(l, b):
            pltpu.async_copy(
                tbl_hbm.at[:, :, pl.ds(col0(l), _CB)], slabs[b], isems[b]
            )

        def compute_and_write(l, b):
            pltpu.make_async_copy(
                tbl_hbm.at[:, :, pl.ds(col0(l), _CB)], slabs[b], isems[b]
            ).wait()

            def row(q, carry):
                for g in range(emb // 16):
                    t16 = tr16 + 2 * g
                    vals0 = plsc.load_gather(
                        slabs[b], [t16, fr16, jnp.full((16,), 0, jnp.int32) + 2 * q]
                    )
                    outs[b][q, pl.ds(g * 16, 16)] = vals0
                    vals1 = plsc.load_gather(
                        slabs[b],
                        [t16, fr16, jnp.full((16,), 1, jnp.int32) + 2 * q],
                    )
                    outs[b][q, pl.ds(emb + g * 16, 16)] = vals1
                return carry

            lax.fori_loop(0, qrows_blk, row, 0)
            pltpu.async_copy(
                outs[b],
                out_hbm.at[pl.ds((wid + l * nw) * qrows_blk, qrows_blk)],
                osems[b],
            )

        def drain_write(b):
            pltpu.make_async_copy(
                outs[b], out_hbm.at[pl.ds(0, qrows_blk)], osems[b]
            ).wait()

        fire(0, 0)
        fire(1, 1)

        def step(t, carry):
            for b in range(2):
                l = t * 2 + b

                @pl.when(l < n_w)
                def _work():
                    compute_and_write(l, b)

                @pl.when(l + 2 < n_w)
                def _refill():
                    drain_write(b)
                    fire(l + 2, b)

            return carry

        lax.fori_loop(0, trips, step, 0)
        drain_write(0)
        drain_write(1)

        if tail_cols:
            tq = tail_cols // 2

            @pl.when(wid == nw - 1)
            def _tail():
                pltpu.sync_copy(tail_hbm, slabs[0])

                def row(q, carry):
                    for g in range(emb // 16):
                        t16 = tr16 + 2 * g
                        vals0 = plsc.load_gather(
                            slabs[0],
                            [t16, fr16, jnp.full((16,), 0, jnp.int32) + 2 * q],
                        )
                        outs[0][q, pl.ds(g * 16, 16)] = vals0
                        vals1 = plsc.load_gather(
                            slabs[0],
                            [t16, fr16, jnp.full((16,), 1, jnp.int32) + 2 * q],
                        )
                        outs[0][q, pl.ds(emb + g * 16, 16)] = vals1
                    return carry

                lax.fori_loop(0, tq, row, 0)
                pltpu.sync_copy(
                    outs[0].at[pl.ds(0, tq)],
                    out_hbm.at[pl.ds(n_glob * qrows_blk, tq)],
                )

    tail = table_t3[:, :, n_glob * _CB :]
    tail_padded = jnp.pad(tail, ((0, 0), (0, 0), (0, _CB - tail_cols)))
    return body(table_t3, tail_padded)


@jax.jit
def _sc_embed(idx_t, table_pairs):
    hist, batch = idx_t.shape
    emb = table_pairs.shape[1] // 2
    tr, fr = emb // 8, 8
    info = plsc.get_sparse_core_info()
    nc, ns = info.num_cores, info.num_subcores
    nw = nc * ns
    n_bblk = batch // _SUB
    assert n_bblk == nw and hist % _NB == 0

    mesh = plsc.VectorSubcoreMesh(core_axis_name="c", subcore_axis_name="s")

    @functools.partial(
        pl.kernel,
        out_type=jax.ShapeDtypeStruct((hist, tr, n_bblk, fr, _SUB), jnp.float32),
        mesh=mesh,
        scratch_types=[
            pltpu.VMEM((hist, _SUB), jnp.int32),
            [pltpu.VMEM((_SUB,), jnp.int32)] * _NB,
            pltpu.VMEM((_SUB,), jnp.int32),
            [pltpu.VMEM((_SUB, 2 * emb), jnp.float32)] * _NB,
            [pltpu.VMEM((tr, fr, _SUB), jnp.float32)] * 2,
            [pltpu.SemaphoreType.DMA] * _NB,
            [pltpu.SemaphoreType.DMA] * 2,
        ],
        compiler_params=pltpu.CompilerParams(
            use_tc_tiling_on_sc=True, needs_layout_passes=False
        ),
    )
    def body(idx_hbm, tbl_hbm, out_hbm, idx_v, q_vs, hv, stages, outbs, gsems, wsems):
        wid = _worker_id()
        pltpu.sync_copy(idx_hbm.at[:, pl.ds(wid * _SUB, _SUB)], idx_v)
        lanes = lax.iota(jnp.int32, 16)

        def fire(h, b):
            for g in range(_SUB // 16):
                sl = pl.ds(g * 16, 16)
                q_vs[b][sl] = lax.shift_right_logical(idx_v[h, sl], 1)
            pltpu.async_copy(tbl_hbm.at[q_vs[b]], stages[b], gsems[b])

        def wait_outb(h, ob):
            pltpu.make_async_copy(
                outbs[ob], out_hbm.at[h, :, wid], wsems[ob]
            ).wait()

        def drain(h, b, ob):
            pltpu.make_async_copy(
                tbl_hbm.at[q_vs[b]], stages[b], gsems[b]
            ).wait()
            for g in range(_SUB // 16):
                sl = pl.ds(g * 16, 16)
                hv[sl] = lax.shift_left(jnp.bitwise_and(idx_v[h, sl], 1), 6)

            def feat(f, carry):
                for g in range(_SUB // 16):
                    p16 = lanes + g * 16
                    cols = hv[pl.ds(g * 16, 16)] + f
                    vals = plsc.load_gather(stages[b], [p16, cols])
                    outbs[ob][f >> 3, f & 7, pl.ds(g * 16, 16)] = vals
                return carry

            lax.fori_loop(0, emb, feat, 0)
            pltpu.async_copy(outbs[ob], out_hbm.at[h, :, wid], wsems[ob])

        for b in range(_NB):
            fire(b, b)

        def step(t, carry):
            for u in range(2):
                for b in range(_NB):
                    k = u * _NB + b
                    h = t * (2 * _NB) + k
                    ob = k % 2

                    @pl.when(h >= 2)
                    def _wait_prev_outb():
                        wait_outb(h - 2, ob)

                    drain(h, b, ob)
                    nxt = h + _NB

                    @pl.when(nxt < hist)
                    def _refill():
                        fire(nxt, b)

            return carry

        lax.fori_loop(0, hist // (2 * _NB), step, 0)
        wait_outb(hist - 2, (2 * _NB - 2) % 2)
        wait_outb(hist - 1, (2 * _NB - 1) % 2)

    return body(idx_t, table_pairs)


def kernel(input_ids, embedding_weight):
    batch, hist = input_ids.shape
    vocab, emb = embedding_weight.shape

    table_pairs = embedding_weight.reshape(vocab // 2, 2 * emb)
    idx_t = input_ids.astype(jnp.int32).T
    out5 = _sc_embed(idx_t, table_pairs)
    return out5.transpose(2, 4, 0, 1, 3).reshape(batch, hist, emb)


# speedup vs baseline: 2.7021x; 1.4270x over previous
import functools

import jax
import jax.numpy as jnp
from jax import lax
from jax.experimental import pallas as pl
from jax.experimental.pallas import tpu as pltpu
from jax.experimental.pallas import tpu_sc as plsc

_SUB = 128
_NB = 5


@functools.partial(jax.jit, static_argnames=("n_rows",))
def _sc_embed(idx_grouped, table_padded, n_rows):
    info = plsc.get_sparse_core_info()
    nc, ns = info.num_cores, info.num_subcores
    nw = nc * ns
    b_per_w = n_rows // nw
    n_sub = b_per_w // _SUB
    two_d = table_padded.shape[1]

    mesh = plsc.VectorSubcoreMesh(core_axis_name="c", subcore_axis_name="s")

    @functools.partial(
        pl.kernel,
        out_type=jax.ShapeDtypeStruct((n_rows, two_d), jnp.float32),
        mesh=mesh,
        scratch_types=[
            pltpu.VMEM((n_sub, _SUB), jnp.int32),
            [pltpu.VMEM((_SUB, two_d), jnp.float32)] * _NB,
            [pltpu.SemaphoreType.DMA] * _NB,
            [pltpu.SemaphoreType.DMA] * _NB,
        ],
        compiler_params=pltpu.CompilerParams(
            use_tc_tiling_on_sc=True, needs_layout_passes=False
        ),
    )
    def body(idx_hbm, tbl_hbm, out_hbm, idx_v, stages, gsems, wsems):
        wid = lax.axis_index("s") * nc + lax.axis_index("c")
        base = wid * b_per_w
        pltpu.sync_copy(idx_hbm.at[wid], idx_v)

        def fire(j, b):
            pltpu.async_copy(tbl_hbm.at[idx_v.at[j]], stages[b], gsems[b])

        def drain(j, b):
            pltpu.make_async_copy(
                tbl_hbm.at[idx_v.at[j]], stages[b], gsems[b]
            ).wait()
            pltpu.async_copy(
                stages[b], out_hbm.at[pl.ds(base + j * _SUB, _SUB)], wsems[b]
            )

        def wait_write(j, b):
            pltpu.make_async_copy(
                stages[b], out_hbm.at[pl.ds(base + j * _SUB, _SUB)], wsems[b]
            ).wait()

        for b in range(_NB):
            fire(b, b)

        def step(t, carry):
            for b in range(_NB):
                j = t * _NB + b
                drain(j, b)
                nxt = j + _NB

                @pl.when(nxt < n_sub)
                def _refill():
                    wait_write(j, b)
                    fire(nxt, b)

            return carry

        lax.fori_loop(0, n_sub // _NB, step, 0)
        for b in range(_NB):
            wait_write(n_sub - _NB + b, b)

    return body(idx_grouped, table_padded)


def kernel(input_ids, embedding_weight):
    batch, hist = input_ids.shape
    vocab, embed_dim = embedding_weight.shape
    n_rows = batch * hist

    info = plsc.get_sparse_core_info()
    nw = info.num_cores * info.num_subcores
    b_per_w = n_rows // nw

    table_padded = jnp.pad(embedding_weight, ((0, 0), (0, 128 - embed_dim)))
    idx_grouped = input_ids.astype(jnp.int32).reshape(nw, b_per_w // _SUB, _SUB)
    out = _sc_embed(idx_grouped, table_padded, n_rows)
    return out[:, :embed_dim].reshape(batch, hist, embed_dim)
